# baseline (speedup 1.0000x reference)
import functools

import jax
import jax.numpy as jnp
from jax import lax
from jax.experimental import pallas as pl
from jax.experimental.pallas import tpu as pltpu
from jax.experimental.pallas import tpu_sc as plsc

B = 16384
F = 26
K = 13
V = 1000000

_info = plsc.get_sparse_core_info()
_NC, _NS = _info.num_cores, _info.num_subcores
_NW = _NC * _NS
_BW = B // _NW
_NIDX = F * _BW
_NCHUNK = _BW // 16

_mesh = plsc.VectorSubcoreMesh(core_axis_name="c", subcore_axis_name="s")


def _tc_numeric_body(numt_ref, w_ref, b_ref, out_ref):
    acc = jnp.full((B,), b_ref[0], dtype=jnp.float32)
    numt = numt_ref[...]
    for k in range(K):
        acc = acc + numt[k, :] * w_ref[0, k]
    out_ref[...] = acc


def _tc_numeric(num_t, num_linear, bias):
    return pl.pallas_call(
        _tc_numeric_body,
        out_shape=jax.ShapeDtypeStruct((B,), jnp.float32),
        in_specs=[
            pl.BlockSpec((K, B), lambda: (0, 0)),
            pl.BlockSpec(memory_space=pltpu.SMEM),
            pl.BlockSpec(memory_space=pltpu.SMEM),
        ],
        out_specs=pl.BlockSpec((B,), lambda: (0,)),
    )(num_t, num_linear, bias)


@functools.partial(
    pl.kernel,
    out_type=jax.ShapeDtypeStruct((B,), jnp.float32),
    mesh=_mesh,
    scratch_types=[
        pltpu.VMEM((_NIDX,), jnp.int32),
        pltpu.VMEM((_NIDX,), jnp.float32),
        pltpu.VMEM((_BW,), jnp.float32),
        pltpu.VMEM((_BW,), jnp.float32),
        pltpu.SemaphoreType.DMA,
        pltpu.SemaphoreType.DMA,
    ],
)
def _sc_gather_sum(idx_hbm, npart_hbm, table_hbm, out_hbm,
                   idx_v, vals_v, npart_v, acc_v, sem, sem2):
    wid = lax.axis_index("s") * _NC + lax.axis_index("c")

    pltpu.sync_copy(idx_hbm.at[wid], idx_v)

    table_flat = table_hbm.at[0]
    _HALF = _NIDX // 2
    _FH = F // 2
    lo = pl.ds(0, _HALF)
    hi = pl.ds(_HALF, _HALF)
    pltpu.make_async_copy(table_flat.at[idx_v.at[lo]], vals_v.at[lo],
                          sem).start()
    pltpu.make_async_copy(table_flat.at[idx_v.at[hi]], vals_v.at[hi],
                          sem2).start()

    pltpu.sync_copy(npart_hbm.at[pl.ds(wid * _BW, _BW)], npart_v)

    pltpu.make_async_copy(table_flat.at[idx_v.at[lo]], vals_v.at[lo],
                          sem).wait()

    def chunk_lo(c, carry):
        acc = npart_v[pl.ds(c * 16, 16)]
        for f in range(_FH):
            acc = acc + vals_v[pl.ds(f * _BW + c * 16, 16)]
        acc_v[pl.ds(c * 16, 16)] = acc
        return carry

    lax.fori_loop(0, _NCHUNK, chunk_lo, 0)

    pltpu.make_async_copy(table_flat.at[idx_v.at[hi]], vals_v.at[hi],
                          sem2).wait()

    def chunk_hi(c, carry):
        acc = acc_v[pl.ds(c * 16, 16)]
        for f in range(_FH, F):
            acc = acc + vals_v[pl.ds(f * _BW + c * 16, 16)]
        acc_v[pl.ds(c * 16, 16)] = acc
        return carry

    lax.fori_loop(0, _NCHUNK, chunk_hi, 0)

    pltpu.sync_copy(acc_v, out_hbm.at[pl.ds(wid * _BW, _BW)])


def kernel(cat_features, num_features, cat_linear_weight, num_linear, bias):
    cat = cat_features.astype(jnp.int32)
    idx_r = (cat.reshape(_NW, _BW, F).transpose(0, 2, 1)
             .reshape(_NW, _NIDX))
    table2 = cat_linear_weight.astype(jnp.float32).reshape(1, V)
    num_t = num_features.astype(jnp.float32).T
    npart = _tc_numeric(num_t, num_linear.astype(jnp.float32),
                        bias.astype(jnp.float32))
    out = _sc_gather_sum(idx_r, npart, table2)
    return out.reshape(B, 1)


# speedup vs baseline: 1.0143x; 1.0143x over previous
import functools

import jax
import jax.numpy as jnp
from jax import lax
from jax.experimental import pallas as pl
from jax.experimental.pallas import tpu as pltpu
from jax.experimental.pallas import tpu_sc as plsc

B = 16384
F = 26
K = 13
V = 1000000

_info = plsc.get_sparse_core_info()
_NC, _NS = _info.num_cores, _info.num_subcores
_NW = _NC * _NS
_BW = B // _NW
_NIDX = F * _BW
_NCHUNK = _BW // 16

_mesh = plsc.VectorSubcoreMesh(core_axis_name="c", subcore_axis_name="s")


@functools.partial(
    pl.kernel,
    out_type=jax.ShapeDtypeStruct((B,), jnp.float32),
    mesh=_mesh,
    scratch_types=[
        pltpu.VMEM((_NIDX,), jnp.int32),
        pltpu.VMEM((_NIDX,), jnp.float32),
        pltpu.VMEM((_BW,), jnp.float32),
        pltpu.SemaphoreType.DMA,
        pltpu.SemaphoreType.DMA,
    ],
)
def _sc_gather_sum(idx_hbm, table_hbm, out_hbm,
                   idx_v, vals_v, acc_v, sem, sem2):
    wid = lax.axis_index("s") * _NC + lax.axis_index("c")

    pltpu.sync_copy(idx_hbm.at[wid], idx_v)

    table_flat = table_hbm.at[0]
    _HALF = _NIDX // 2
    _FH = F // 2
    lo = pl.ds(0, _HALF)
    hi = pl.ds(_HALF, _HALF)
    pltpu.make_async_copy(table_flat.at[idx_v.at[lo]], vals_v.at[lo],
                          sem).start()
    pltpu.make_async_copy(table_flat.at[idx_v.at[hi]], vals_v.at[hi],
                          sem2).start()

    pltpu.make_async_copy(table_flat.at[idx_v.at[lo]], vals_v.at[lo],
                          sem).wait()

    def chunk_lo(c, carry):
        acc = vals_v[pl.ds(c * 16, 16)]
        for f in range(1, _FH):
            acc = acc + vals_v[pl.ds(f * _BW + c * 16, 16)]
        acc_v[pl.ds(c * 16, 16)] = acc
        return carry

    lax.fori_loop(0, _NCHUNK, chunk_lo, 0)

    pltpu.make_async_copy(table_flat.at[idx_v.at[hi]], vals_v.at[hi],
                          sem2).wait()

    def chunk_hi(c, carry):
        acc = acc_v[pl.ds(c * 16, 16)]
        for f in range(_FH, F):
            acc = acc + vals_v[pl.ds(f * _BW + c * 16, 16)]
        acc_v[pl.ds(c * 16, 16)] = acc
        return carry

    lax.fori_loop(0, _NCHUNK, chunk_hi, 0)

    pltpu.sync_copy(acc_v, out_hbm.at[pl.ds(wid * _BW, _BW)])


def _tc_combine_body(cat_ref, numt_ref, w_ref, b_ref, out_ref):
    acc = cat_ref[...] + b_ref[0]
    numt = numt_ref[...]
    for k in range(K):
        acc = acc + numt[k, :] * w_ref[0, k]
    out_ref[...] = acc


_BBLK = B


def _tc_combine(cat_sums, num_t, num_linear, bias):
    return pl.pallas_call(
        _tc_combine_body,
        out_shape=jax.ShapeDtypeStruct((B,), jnp.float32),
        grid=(B // _BBLK,),
        in_specs=[
            pl.BlockSpec((_BBLK,), lambda i: (i,)),
            pl.BlockSpec((K, _BBLK), lambda i: (0, i)),
            pl.BlockSpec(memory_space=pltpu.SMEM),
            pl.BlockSpec(memory_space=pltpu.SMEM),
        ],
        out_specs=pl.BlockSpec((_BBLK,), lambda i: (i,)),
    )(cat_sums, num_t, num_linear, bias)


def kernel(cat_features, num_features, cat_linear_weight, num_linear, bias):
    cat = cat_features.astype(jnp.int32)
    idx_r = (cat.reshape(_NW, _BW, F).transpose(0, 2, 1)
             .reshape(_NW, _NIDX))
    table2 = cat_linear_weight.astype(jnp.float32).reshape(1, V)
    cat_sums = _sc_gather_sum(idx_r, table2)
    num_t = num_features.astype(jnp.float32).T
    out = _tc_combine(cat_sums, num_t, num_linear.astype(jnp.float32),
                      bias.astype(jnp.float32))
    return out.reshape(B, 1)


# speedup vs baseline: 1.0161x; 1.0018x over previous
import functools

import jax
import jax.numpy as jnp
from jax import lax
from jax.experimental import pallas as pl
from jax.experimental.pallas import tpu as pltpu
from jax.experimental.pallas import tpu_sc as plsc

B = 16384
F = 26
K = 13
V = 1000000

_info = plsc.get_sparse_core_info()
_NC, _NS = _info.num_cores, _info.num_subcores
_NW = _NC * _NS
_BW = B // _NW
_NIDX = F * _BW
_NCHUNK = _BW // 16

_mesh = plsc.VectorSubcoreMesh(core_axis_name="c", subcore_axis_name="s")


@functools.partial(
    pl.kernel,
    out_type=jax.ShapeDtypeStruct((B,), jnp.float32),
    mesh=_mesh,
    scratch_types=[
        pltpu.VMEM((_NIDX,), jnp.int32),
        pltpu.VMEM((_NIDX,), jnp.float32),
        pltpu.VMEM((_BW,), jnp.float32),
        [pltpu.SemaphoreType.DMA] * 4,
    ],
)
def _sc_gather_sum(idx_hbm, table_hbm, out_hbm,
                   idx_v, vals_v, acc_v, sems):
    wid = lax.axis_index("s") * _NC + lax.axis_index("c")

    pltpu.sync_copy(idx_hbm.at[wid], idx_v)

    table_flat = table_hbm.at[0]
    _FQ = (0, 6, 13, 19, F)
    parts = []
    for q in range(4):
        sl = pl.ds(_FQ[q] * _BW, (_FQ[q + 1] - _FQ[q]) * _BW)
        parts.append(pltpu.make_async_copy(
            table_flat.at[idx_v.at[sl]], vals_v.at[sl], sems[q]))
    for q in range(4):
        parts[q].start()

    for q in range(4):
        parts[q].wait()

        def chunk(c, carry, q=q):
            if q == 0:
                acc = vals_v[pl.ds(c * 16, 16)]
                f0 = 1
            else:
                acc = acc_v[pl.ds(c * 16, 16)]
                f0 = _FQ[q]
            for f in range(f0, _FQ[q + 1]):
                acc = acc + vals_v[pl.ds(f * _BW + c * 16, 16)]
            acc_v[pl.ds(c * 16, 16)] = acc
            return carry

        lax.fori_loop(0, _NCHUNK, chunk, 0)

    pltpu.sync_copy(acc_v, out_hbm.at[pl.ds(wid * _BW, _BW)])


def _tc_combine_body(cat_ref, numt_ref, w_ref, b_ref, out_ref):
    acc = cat_ref[...] + b_ref[0]
    numt = numt_ref[...]
    for k in range(K):
        acc = acc + numt[k, :] * w_ref[0, k]
    out_ref[...] = acc


_BBLK = B


def _tc_combine(cat_sums, num_t, num_linear, bias):
    return pl.pallas_call(
        _tc_combine_body,
        out_shape=jax.ShapeDtypeStruct((B,), jnp.float32),
        grid=(B // _BBLK,),
        in_specs=[
            pl.BlockSpec((_BBLK,), lambda i: (i,)),
            pl.BlockSpec((K, _BBLK), lambda i: (0, i)),
            pl.BlockSpec(memory_space=pltpu.SMEM),
            pl.BlockSpec(memory_space=pltpu.SMEM),
        ],
        out_specs=pl.BlockSpec((_BBLK,), lambda i: (i,)),
    )(cat_sums, num_t, num_linear, bias)


def kernel(cat_features, num_features, cat_linear_weight, num_linear, bias):
    cat = cat_features.astype(jnp.int32)
    idx_r = (cat.reshape(_NW, _BW, F).transpose(0, 2, 1)
             .reshape(_NW, _NIDX))
    table2 = cat_linear_weight.astype(jnp.float32).reshape(1, V)
    cat_sums = _sc_gather_sum(idx_r, table2)
    num_t = num_features.astype(jnp.float32).T
    out = _tc_combine(cat_sums, num_t, num_linear.astype(jnp.float32),
                      bias.astype(jnp.float32))
    return out.reshape(B, 1)


# speedup vs baseline: 1.0282x; 1.0119x over previous
import functools

import jax
import jax.numpy as jnp
from jax import lax
from jax.experimental import pallas as pl
from jax.experimental.pallas import tpu as pltpu
from jax.experimental.pallas import tpu_sc as plsc

B = 16384
F = 26
K = 13
V = 1000000

_info = plsc.get_sparse_core_info()
_NC, _NS = _info.num_cores, _info.num_subcores
_NW = _NC * _NS
_BW = B // _NW
_NIDX = F * _BW
_NCHUNK = _BW // 16

_mesh = plsc.VectorSubcoreMesh(core_axis_name="c", subcore_axis_name="s")


@functools.partial(
    pl.kernel,
    out_type=jax.ShapeDtypeStruct((B,), jnp.float32),
    mesh=_mesh,
    scratch_types=[
        pltpu.VMEM((_NIDX,), jnp.int32),
        pltpu.VMEM((_NIDX,), jnp.float32),
        pltpu.VMEM((_BW,), jnp.float32),
        [pltpu.SemaphoreType.DMA] * 4,
    ],
)
def _sc_gather_sum(idx_hbm, table_hbm, out_hbm,
                   idx_v, vals_v, acc_v, sems):
    wid = lax.axis_index("s") * _NC + lax.axis_index("c")

    pltpu.sync_copy(idx_hbm.at[wid], idx_v)

    table_flat = table_hbm.at[0]
    _FQ = (0, 6, 13, 19, F)
    parts = []
    for q in range(4):
        sl = pl.ds(_FQ[q] * _BW, (_FQ[q + 1] - _FQ[q]) * _BW)
        parts.append(pltpu.make_async_copy(
            table_flat.at[idx_v.at[sl]], vals_v.at[sl], sems[q]))
    for q in range(4):
        parts[q].start()

    for q in range(4):
        parts[q].wait()

        def chunk(c, carry, q=q):
            if q == 0:
                acc = vals_v[pl.ds(c * 16, 16)]
                f0 = 1
            else:
                acc = acc_v[pl.ds(c * 16, 16)]
                f0 = _FQ[q]
            for f in range(f0, _FQ[q + 1]):
                acc = acc + vals_v[pl.ds(f * _BW + c * 16, 16)]
            acc_v[pl.ds(c * 16, 16)] = acc
            return carry

        lax.fori_loop(0, _NCHUNK, chunk, 0)

    pltpu.sync_copy(acc_v, out_hbm.at[pl.ds(wid * _BW, _BW)])


_R2 = 128


def _tc_combine_body(cat_ref, numt_ref, w_ref, b_ref, out_ref):
    acc = cat_ref[...] + b_ref[0]
    for k in range(K):
        acc = acc + numt_ref[k] * w_ref[0, k]
    out_ref[...] = acc


def _tc_combine(cat_sums, num_t, num_linear, bias):
    return pl.pallas_call(
        _tc_combine_body,
        out_shape=jax.ShapeDtypeStruct((_R2, _R2), jnp.float32),
        in_specs=[
            pl.BlockSpec((_R2, _R2), lambda: (0, 0)),
            pl.BlockSpec((K, _R2, _R2), lambda: (0, 0, 0)),
            pl.BlockSpec(memory_space=pltpu.SMEM),
            pl.BlockSpec(memory_space=pltpu.SMEM),
        ],
        out_specs=pl.BlockSpec((_R2, _R2), lambda: (0, 0)),
    )(cat_sums.reshape(_R2, _R2), num_t.reshape(K, _R2, _R2),
      num_linear, bias)


def kernel(cat_features, num_features, cat_linear_weight, num_linear, bias):
    cat = cat_features.astype(jnp.int32)
    idx_r = (cat.reshape(_NW, _BW, F).transpose(0, 2, 1)
             .reshape(_NW, _NIDX))
    table2 = cat_linear_weight.astype(jnp.float32).reshape(1, V)
    cat_sums = _sc_gather_sum(idx_r, table2)
    num_t = num_features.astype(jnp.float32).T
    out = _tc_combine(cat_sums, num_t, num_linear.astype(jnp.float32),
                      bias.astype(jnp.float32))
    return out.reshape(B, 1)
